# baseline (speedup 1.0000x reference)
import functools

import jax
import jax.numpy as jnp
from jax import lax
from jax.experimental import pallas as pl
from jax.experimental.pallas import tpu as pltpu
from jax.experimental.pallas import tpu_sc as plsc

_VOCAB = 100000
_EMB = 64
_SEQ = 8192
_BATCH = 4
_TOT = _BATCH * _SEQ
_NC = 2
_NS = 16
_NW = _NC * _NS
_PER_W = _TOT // _NW
_CHUNK = 128
_NCH = _PER_W // _CHUNK
_LANES = 16


def _emb_body(x_hbm, tok_hbm, pos_hbm, out_hbm, idx_v, pos_v, rows_v, sem):
    cid = lax.axis_index("c")
    sid = lax.axis_index("s")
    wid = sid * _NC + cid
    base = wid * _PER_W
    pos_base = lax.rem(base, _SEQ)

    pltpu.sync_copy(x_hbm.at[pl.ds(wid * _NCH, _NCH)], idx_v)
    pltpu.sync_copy(pos_hbm.at[pl.ds(pos_base, _PER_W)], pos_v)

    for j in range(_NCH):
        pltpu.async_copy(tok_hbm.at[idx_v.at[j]], rows_v, sem).wait()

        def add_row(r, _):
            for g in range(_EMB // _LANES):
                sl = pl.ds(g * _LANES, _LANES)
                rows_v[r, sl] = rows_v[r, sl] + pos_v[j * _CHUNK + r, sl]
            return 0

        lax.fori_loop(0, _CHUNK, add_row, 0)

        pltpu.sync_copy(rows_v, out_hbm.at[pl.ds(base + j * _CHUNK, _CHUNK)])


@jax.jit
def _emb(xi, token_table, pos_table):
    mesh = plsc.VectorSubcoreMesh(core_axis_name="c", subcore_axis_name="s")
    run = functools.partial(
        pl.kernel,
        mesh=mesh,
        out_type=jax.ShapeDtypeStruct((_TOT, _EMB), jnp.float32),
        scratch_types=[
            pltpu.VMEM((_NCH, _CHUNK), jnp.int32),
            pltpu.VMEM((_PER_W, _EMB), jnp.float32),
            pltpu.VMEM((_CHUNK, _EMB), jnp.float32),
            pltpu.SemaphoreType.DMA,
        ],
        compiler_params=pltpu.CompilerParams(use_tc_tiling_on_sc=False),
    )(_emb_body)
    return run(xi, token_table, pos_table)


def kernel(x, token_table, pos_table):
    xi = x.astype(jnp.int32).reshape(_NW * _NCH, _CHUNK)
    out = _emb(xi, token_table, pos_table)
    return out.reshape(_BATCH, _SEQ, _EMB)


# speedup vs baseline: 1.0420x; 1.0420x over previous
import functools

import jax
import jax.numpy as jnp
from jax import lax
from jax.experimental import pallas as pl
from jax.experimental.pallas import tpu as pltpu
from jax.experimental.pallas import tpu_sc as plsc

_VOCAB = 100000
_EMB = 64
_SEQ = 8192
_BATCH = 4
_TOT = _BATCH * _SEQ
_NC = 2
_NS = 16
_NW = _NC * _NS
_PER_W = _TOT // _NW
_CHUNK = 128
_NCH = _PER_W // _CHUNK
_LANES = 16
_NB = 4
_DEPTH = 2


def _emb_body(x_hbm, tok_hbm, pos_hbm, out_hbm, idx_v, pos_v, rows_v,
              gsem, osem, psem):
    cid = lax.axis_index("c")
    sid = lax.axis_index("s")
    wid = sid * _NC + cid
    base = wid * _PER_W
    pos_base = lax.rem(base, _SEQ)

    pltpu.sync_copy(x_hbm.at[pl.ds(wid * _NCH, _NCH)], idx_v)
    pos_cp = pltpu.async_copy(pos_hbm.at[pl.ds(pos_base, _PER_W)], pos_v, psem)

    gathers = {}
    outs = {}
    for j in range(-_DEPTH, _NCH):
        f = j + _DEPTH
        if 0 <= f < _NCH:
            if f - _NB >= 0:
                outs[f - _NB].wait()
            gathers[f] = pltpu.async_copy(
                tok_hbm.at[idx_v.at[f]], rows_v.at[f % _NB], gsem)
        if j < 0:
            continue

        gathers[j].wait()
        if j == 0:
            pos_cp.wait()

        slot = j % _NB

        def add_row(r, _):
            for g in range(_EMB // _LANES):
                sl = pl.ds(g * _LANES, _LANES)
                rows_v[slot, r, sl] = rows_v[slot, r, sl] + pos_v[j * _CHUNK + r, sl]
            return 0

        lax.fori_loop(0, _CHUNK, add_row, 0, unroll=4)

        outs[j] = pltpu.async_copy(
            rows_v.at[slot], out_hbm.at[pl.ds(base + j * _CHUNK, _CHUNK)], osem)

    for j in range(_NCH - _NB, _NCH):
        if j >= 0:
            outs[j].wait()


@jax.jit
def _emb(xi, token_table, pos_table):
    mesh = plsc.VectorSubcoreMesh(core_axis_name="c", subcore_axis_name="s")
    run = functools.partial(
        pl.kernel,
        mesh=mesh,
        out_type=jax.ShapeDtypeStruct((_TOT, _EMB), jnp.float32),
        scratch_types=[
            pltpu.VMEM((_NCH, _CHUNK), jnp.int32),
            pltpu.VMEM((_PER_W, _EMB), jnp.float32),
            pltpu.VMEM((_NB, _CHUNK, _EMB), jnp.float32),
            pltpu.SemaphoreType.DMA,
            pltpu.SemaphoreType.DMA,
            pltpu.SemaphoreType.DMA,
        ],
        compiler_params=pltpu.CompilerParams(use_tc_tiling_on_sc=False),
    )(_emb_body)
    return run(xi, token_table, pos_table)


def kernel(x, token_table, pos_table):
    xi = x.astype(jnp.int32).reshape(_NW * _NCH, _CHUNK)
    out = _emb(xi, token_table, pos_table)
    return out.reshape(_BATCH, _SEQ, _EMB)
